# baseline (speedup 1.0000x reference)
import jax
import jax.numpy as jnp
from jax.experimental import pallas as pl

_PATCH = 16
_P = 0.5
_VALUE = 0.0


def _static_mask2d(h, w):
    nps = h // _PATCH
    npatch = nps * nps
    nmask = int(_P * npatch)
    base = jnp.concatenate([
        jnp.ones((nmask,), dtype=jnp.float32),
        jnp.zeros((npatch - nmask,), dtype=jnp.float32),
    ])
    perm = jax.random.permutation(jax.random.key(42), npatch)
    masked = base[perm].reshape(nps, nps)
    return jnp.kron(masked, jnp.ones((_PATCH, _PATCH), dtype=jnp.float32))


def _erase_body(m_ref, x_ref, o_ref):
    o_ref[...] = jnp.where(m_ref[...][None] != 0.0,
                           jnp.float32(_VALUE), x_ref[...])


def kernel(img):
    c, h, w = img.shape
    m = _static_mask2d(h, w)
    cb = 2
    return pl.pallas_call(
        _erase_body,
        grid=(c // cb,),
        in_specs=[
            pl.BlockSpec((h, w), lambda i: (0, 0)),
            pl.BlockSpec((cb, h, w), lambda i: (i, 0, 0)),
        ],
        out_specs=pl.BlockSpec((cb, h, w), lambda i: (i, 0, 0)),
        out_shape=jax.ShapeDtypeStruct((c, h, w), img.dtype),
    )(m, img)


# speedup vs baseline: 1.0924x; 1.0924x over previous
import jax
import jax.numpy as jnp
from jax.experimental import pallas as pl

_PATCH = 16
_P = 0.5
_VALUE = 0.0


def _static_mask2d(h, w):
    nps = h // _PATCH
    npatch = nps * nps
    nmask = int(_P * npatch)
    base = jnp.concatenate([
        jnp.ones((nmask,), dtype=jnp.float32),
        jnp.zeros((npatch - nmask,), dtype=jnp.float32),
    ])
    perm = jax.random.permutation(jax.random.key(42), npatch)
    masked = base[perm].reshape(nps, nps)
    return jnp.kron(masked, jnp.ones((_PATCH, _PATCH), dtype=jnp.float32))


def _erase_body(m_ref, x_ref, o_ref):
    o_ref[...] = jnp.where(m_ref[...][None] != 0.0,
                           jnp.float32(_VALUE), x_ref[...])


def kernel(img):
    c, h, w = img.shape
    m = _static_mask2d(h, w)
    cb = 8
    return pl.pallas_call(
        _erase_body,
        grid=(c // cb,),
        in_specs=[
            pl.BlockSpec((h, w), lambda i: (0, 0)),
            pl.BlockSpec((cb, h, w), lambda i: (i, 0, 0)),
        ],
        out_specs=pl.BlockSpec((cb, h, w), lambda i: (i, 0, 0)),
        out_shape=jax.ShapeDtypeStruct((c, h, w), img.dtype),
    )(m, img)
